# baseline (speedup 1.0000x reference)
import functools

import jax
import jax.numpy as jnp
from jax import lax
from jax.experimental import pallas as pl
from jax.experimental.pallas import tpu as pltpu
from jax.experimental.pallas import tpu_sc as plsc

_N_NODES = 10000
_N_EDGES = 320000
_D = 128

_NW = 32
_UNIT = 40
_SUP = 200
_NU = _SUP // _UNIT

_MM_BLK = 1000
_K = 5
_SEG = _N_EDGES // _K


def _make_gather_add(seg_base, seg_edges, interpret):
  epw = seg_edges // _NW
  nsup = epw // _SUP
  assert nsup * _SUP == epw and epw % 8 == 0

  mesh = plsc.VectorSubcoreMesh(
      core_axis_name="c", subcore_axis_name="s", num_cores=2, num_subcores=16)

  _DW = _D // 2

  @functools.partial(
      pl.kernel,
      out_type=jax.ShapeDtypeStruct((seg_edges, _DW), jnp.int32),
      mesh=mesh,
      scratch_types=[
          pltpu.VMEM((epw,), jnp.int32),
          pltpu.VMEM((epw,), jnp.int32),
          pltpu.VMEM((2, _SUP, _DW), jnp.int32),
          pltpu.VMEM((2, _SUP, _DW), jnp.int32),
          pltpu.SemaphoreType.DMA,
          pltpu.SemaphoreType.DMA,
          pltpu.SemaphoreType.DMA,
          pltpu.SemaphoreType.DMA,
          pltpu.SemaphoreType.DMA,
          pltpu.SemaphoreType.DMA,
      ],
      compiler_params=pltpu.CompilerParams(
          use_tc_tiling_on_sc=False, needs_layout_passes=False),
      interpret=interpret,
  )
  def gather_add(pr_hbm, ps_hbm, recv_hbm, send_hbm, out_hbm,
                 idx_r, idx_s, rows_r, rows_s,
                 sem_r0, sem_s0, sem_r1, sem_s1, sem_o0, sem_o1):
    wid = lax.axis_index("s") * 2 + lax.axis_index("c")
    out_base = wid * epw
    in_base = seg_base + out_base
    sem_r = (sem_r0, sem_r1)
    sem_s = (sem_s0, sem_s1)
    sem_o = (sem_o0, sem_o1)

    pltpu.sync_copy(recv_hbm.at[pl.ds(in_base, epw)], idx_r)
    pltpu.sync_copy(send_hbm.at[pl.ds(in_base, epw)], idx_s)

    def fire(b, si):
      for u in range(_NU):
        o = si * _SUP + u * _UNIT
        dst = pl.ds(u * _UNIT, _UNIT)
        pltpu.async_copy(pr_hbm.at[idx_r.at[pl.ds(o, _UNIT)]],
                         rows_r.at[b, dst], sem_r[b])
        pltpu.async_copy(ps_hbm.at[idx_s.at[pl.ds(o, _UNIT)]],
                         rows_s.at[b, dst], sem_s[b])

    def drain(b, si):
      for u in range(_NU):
        o = si * _SUP + u * _UNIT
        dst = pl.ds(u * _UNIT, _UNIT)
        pltpu.make_async_copy(pr_hbm.at[idx_r.at[pl.ds(o, _UNIT)]],
                              rows_r.at[b, dst], sem_r[b]).wait()
        pltpu.make_async_copy(ps_hbm.at[idx_s.at[pl.ds(o, _UNIT)]],
                              rows_s.at[b, dst], sem_s[b]).wait()

    def consume(b, si):
      drain(b, si)

      def add_row(r, c2):
        for j in range(_DW // 16):
          sl = pl.ds(j * 16, 16)
          a = plsc.bitcast(rows_r[b, r, sl], jnp.bfloat16)
          s = plsc.bitcast(rows_s[b, r, sl], jnp.bfloat16)
          rows_r[b, r, sl] = plsc.bitcast(a + s, jnp.int32)
        return c2

      lax.fori_loop(0, _SUP, add_row, 0)
      pltpu.async_copy(rows_r.at[b],
                       out_hbm.at[pl.ds(out_base + si * _SUP, _SUP)], sem_o[b])

    def wait_out(b, si):
      pltpu.make_async_copy(rows_r.at[b],
                            out_hbm.at[pl.ds(out_base + si * _SUP, _SUP)],
                            sem_o[b]).wait()

    fire(0, 0)
    fire(1, 1)

    def pair_body(i, carry):
      s0 = 2 * i
      consume(0, s0)

      @pl.when(s0 + 2 < nsup)
      def _():
        wait_out(0, s0)
        fire(0, s0 + 2)

      consume(1, s0 + 1)

      @pl.when(s0 + 3 < nsup)
      def _():
        wait_out(1, s0 + 1)
        fire(1, s0 + 3)

      return carry

    lax.fori_loop(0, nsup // 2, pair_body, 0)
    if nsup % 2:
      consume(0, nsup - 1)
      wait_out(0, nsup - 1)
      wait_out(1, nsup - 2)
    else:
      wait_out(0, nsup - 2)
      wait_out(1, nsup - 1)

  return gather_add


def _build(interpret: bool = False):
  def _proj_body(nodes_ref, wr_ref, ws_ref, pr_ref, ps_ref):
    n = nodes_ref[...]
    pr_ref[...] = jnp.dot(
        n, wr_ref[...], preferred_element_type=jnp.float32).astype(jnp.bfloat16)
    ps_ref[...] = jnp.dot(
        n, ws_ref[...], preferred_element_type=jnp.float32).astype(jnp.bfloat16)

  proj = pl.pallas_call(
      _proj_body,
      out_shape=(
          jax.ShapeDtypeStruct((_N_NODES, _D), jnp.bfloat16),
          jax.ShapeDtypeStruct((_N_NODES, _D), jnp.bfloat16),
      ),
      interpret=interpret,
  )

  scs = [_make_gather_add(k * _SEG, _SEG, interpret) for k in range(_K)]

  blocks = _SEG // _MM_BLK

  def _mm_body(e_ref, w_ref, g_ref, o_ref):
    o_ref[...] = (
        jnp.dot(e_ref[...], w_ref[...], preferred_element_type=jnp.float32)
        + g_ref[...].astype(jnp.float32)
    )

  def _mm_chain_body(e_ref, w_ref, g_ref, prev_ref, o_ref):
    del prev_ref
    o_ref[...] = (
        jnp.dot(e_ref[...], w_ref[...], preferred_element_type=jnp.float32)
        + g_ref[...].astype(jnp.float32)
    )

  mms = []
  for k in range(_K):
    e_spec = pl.BlockSpec((_MM_BLK, _D), lambda i, k0=k: (i + k0 * blocks, 0))
    w_spec = pl.BlockSpec((_D, _D), lambda i: (0, 0))
    g_spec = pl.BlockSpec((_MM_BLK, _D), lambda i: (i, 0))
    o_spec = pl.BlockSpec((_MM_BLK, _D), lambda i, k0=k: (i + k0 * blocks, 0))
    if k == 0:
      mms.append(pl.pallas_call(
          _mm_body,
          grid=(blocks,),
          in_specs=[e_spec, w_spec, g_spec],
          out_specs=o_spec,
          out_shape=jax.ShapeDtypeStruct((_N_EDGES, _D), jnp.float32),
          interpret=interpret,
      ))
    else:
      mms.append(pl.pallas_call(
          _mm_chain_body,
          grid=(blocks,),
          in_specs=[e_spec, w_spec, g_spec,
                    pl.BlockSpec(memory_space=pl.ANY)],
          out_specs=o_spec,
          out_shape=jax.ShapeDtypeStruct((_N_EDGES, _D), jnp.float32),
          input_output_aliases={3: 0},
          interpret=interpret,
      ))

  return proj, scs, mms


_CACHE = []


def kernel(nodes, edges, receivers, senders, W):
  if not _CACHE:
    _CACHE.append(_build(False))
  proj, scs, mms = _CACHE[0]
  we = W[:_D]
  wr = W[_D:2 * _D]
  ws = W[2 * _D:]
  pr, ps = proj(nodes, wr, ws)
  pr32 = lax.bitcast_convert_type(pr.reshape(_N_NODES, _D // 2, 2), jnp.int32)
  ps32 = lax.bitcast_convert_type(ps.reshape(_N_NODES, _D // 2, 2), jnp.int32)
  recv = receivers.astype(jnp.int32)
  send = senders.astype(jnp.int32)
  gs = [
      jnp.reshape(lax.bitcast_convert_type(sc(pr32, ps32, recv, send),
                                           jnp.bfloat16), (_SEG, _D))
      for sc in scs
  ]
  out = mms[0](edges, we, gs[0])
  for k in range(1, _K):
    out = mms[k](edges, we, gs[k], out)
  return out


# speedup vs baseline: 3.4636x; 3.4636x over previous
import functools

import jax
import jax.numpy as jnp
from jax import lax
from jax.experimental import pallas as pl
from jax.experimental.pallas import tpu as pltpu
from jax.experimental.pallas import tpu_sc as plsc

_N_NODES = 10000
_N_EDGES = 320000
_D = 128

_NW = 32
_UNIT = 40
_SUP = 200
_NU = _SUP // _UNIT

_MM_BLK = 2000
_K = 5
_SEG = _N_EDGES // _K


def _make_gather_add(seg_base, seg_edges, interpret):
  epw = seg_edges // _NW
  nsup = epw // _SUP
  assert nsup * _SUP == epw and epw % 8 == 0

  mesh = plsc.VectorSubcoreMesh(
      core_axis_name="c", subcore_axis_name="s", num_cores=2, num_subcores=16)

  @functools.partial(
      pl.kernel,
      out_type=jax.ShapeDtypeStruct((seg_edges, _D), jnp.float32),
      mesh=mesh,
      scratch_types=[
          pltpu.VMEM((epw,), jnp.int32),
          pltpu.VMEM((epw,), jnp.int32),
          pltpu.VMEM((2, _SUP, _D), jnp.float32),
          pltpu.VMEM((2, _SUP, _D), jnp.float32),
          pltpu.SemaphoreType.DMA,
          pltpu.SemaphoreType.DMA,
          pltpu.SemaphoreType.DMA,
          pltpu.SemaphoreType.DMA,
          pltpu.SemaphoreType.DMA,
          pltpu.SemaphoreType.DMA,
      ],
      interpret=interpret,
  )
  def gather_add(pr_hbm, ps_hbm, recv_hbm, send_hbm, out_hbm,
                 idx_r, idx_s, rows_r, rows_s,
                 sem_r0, sem_s0, sem_r1, sem_s1, sem_o0, sem_o1):
    wid = lax.axis_index("s") * 2 + lax.axis_index("c")
    out_base = wid * epw
    in_base = seg_base + out_base
    sem_r = (sem_r0, sem_r1)
    sem_s = (sem_s0, sem_s1)
    sem_o = (sem_o0, sem_o1)

    pltpu.sync_copy(recv_hbm.at[pl.ds(in_base, epw)], idx_r)
    pltpu.sync_copy(send_hbm.at[pl.ds(in_base, epw)], idx_s)

    def fire(b, si):
      for u in range(_NU):
        o = si * _SUP + u * _UNIT
        dst = pl.ds(u * _UNIT, _UNIT)
        pltpu.async_copy(pr_hbm.at[idx_r.at[pl.ds(o, _UNIT)]],
                         rows_r.at[b, dst], sem_r[b])
        pltpu.async_copy(ps_hbm.at[idx_s.at[pl.ds(o, _UNIT)]],
                         rows_s.at[b, dst], sem_s[b])

    def drain(b, si):
      for u in range(_NU):
        o = si * _SUP + u * _UNIT
        dst = pl.ds(u * _UNIT, _UNIT)
        pltpu.make_async_copy(pr_hbm.at[idx_r.at[pl.ds(o, _UNIT)]],
                              rows_r.at[b, dst], sem_r[b]).wait()
        pltpu.make_async_copy(ps_hbm.at[idx_s.at[pl.ds(o, _UNIT)]],
                              rows_s.at[b, dst], sem_s[b]).wait()

    def consume(b, si):
      drain(b, si)

      def add_row(r, c2):
        for j in range(_D // 16):
          sl = pl.ds(j * 16, 16)
          plsc.addupdate(rows_r.at[b, r, sl], rows_s[b, r, sl])
        return c2

      lax.fori_loop(0, _SUP, add_row, 0)
      pltpu.async_copy(rows_r.at[b],
                       out_hbm.at[pl.ds(out_base + si * _SUP, _SUP)], sem_o[b])

    def wait_out(b, si):
      pltpu.make_async_copy(rows_r.at[b],
                            out_hbm.at[pl.ds(out_base + si * _SUP, _SUP)],
                            sem_o[b]).wait()

    fire(0, 0)
    fire(1, 1)

    def pair_body(i, carry):
      s0 = 2 * i
      consume(0, s0)

      @pl.when(s0 + 2 < nsup)
      def _():
        wait_out(0, s0)
        fire(0, s0 + 2)

      consume(1, s0 + 1)

      @pl.when(s0 + 3 < nsup)
      def _():
        wait_out(1, s0 + 1)
        fire(1, s0 + 3)

      return carry

    lax.fori_loop(0, nsup // 2, pair_body, 0)
    if nsup % 2:
      consume(0, nsup - 1)
      wait_out(0, nsup - 1)
      wait_out(1, nsup - 2)
    else:
      wait_out(0, nsup - 2)
      wait_out(1, nsup - 1)

  return gather_add


def _build(interpret: bool = False):
  def _proj_body(nodes_ref, wr_ref, ws_ref, pr_ref, ps_ref):
    n = nodes_ref[...]
    pr_ref[...] = jnp.dot(n, wr_ref[...], preferred_element_type=jnp.float32)
    ps_ref[...] = jnp.dot(n, ws_ref[...], preferred_element_type=jnp.float32)

  proj = pl.pallas_call(
      _proj_body,
      out_shape=(
          jax.ShapeDtypeStruct((_N_NODES, _D), jnp.float32),
          jax.ShapeDtypeStruct((_N_NODES, _D), jnp.float32),
      ),
      interpret=interpret,
  )

  scs = [_make_gather_add(k * _SEG, _SEG, interpret) for k in range(_K)]

  blocks = _SEG // _MM_BLK

  def _mm_body(e_ref, w_ref, g_ref, o_ref):
    o_ref[...] = (
        jnp.dot(e_ref[...], w_ref[...], preferred_element_type=jnp.float32)
        + g_ref[...]
    )

  def _mm_chain_body(e_ref, w_ref, g_ref, prev_ref, o_ref):
    del prev_ref
    o_ref[...] = (
        jnp.dot(e_ref[...], w_ref[...], preferred_element_type=jnp.float32)
        + g_ref[...]
    )

  mms = []
  for k in range(_K):
    e_spec = pl.BlockSpec((_MM_BLK, _D), lambda i, k0=k: (i + k0 * blocks, 0))
    w_spec = pl.BlockSpec((_D, _D), lambda i: (0, 0))
    g_spec = pl.BlockSpec((_MM_BLK, _D), lambda i: (i, 0))
    o_spec = pl.BlockSpec((_MM_BLK, _D), lambda i, k0=k: (i + k0 * blocks, 0))
    if k == 0:
      mms.append(pl.pallas_call(
          _mm_body,
          grid=(blocks,),
          in_specs=[e_spec, w_spec, g_spec],
          out_specs=o_spec,
          out_shape=jax.ShapeDtypeStruct((_N_EDGES, _D), jnp.float32),
          interpret=interpret,
      ))
    else:
      mms.append(pl.pallas_call(
          _mm_chain_body,
          grid=(blocks,),
          in_specs=[e_spec, w_spec, g_spec,
                    pl.BlockSpec(memory_space=pl.ANY)],
          out_specs=o_spec,
          out_shape=jax.ShapeDtypeStruct((_N_EDGES, _D), jnp.float32),
          input_output_aliases={3: 0},
          interpret=interpret,
      ))

  return proj, scs, mms


_CACHE = []


def kernel(nodes, edges, receivers, senders, W):
  if not _CACHE:
    _CACHE.append(_build(False))
  proj, scs, mms = _CACHE[0]
  we = W[:_D]
  wr = W[_D:2 * _D]
  ws = W[2 * _D:]
  pr, ps = proj(nodes, wr, ws)
  recv = receivers.astype(jnp.int32)
  send = senders.astype(jnp.int32)
  gs = [sc(pr, ps, recv, send) for sc in scs]
  out = mms[0](edges, we, gs[0])
  for k in range(1, _K):
    out = mms[k](edges, we, gs[k], out)
  return out


# speedup vs baseline: 3.5383x; 1.0216x over previous
import functools

import jax
import jax.numpy as jnp
from jax import lax
from jax.experimental import pallas as pl
from jax.experimental.pallas import tpu as pltpu
from jax.experimental.pallas import tpu_sc as plsc

_N_NODES = 10000
_N_EDGES = 320000
_D = 128

_NW = 32
_UNIT = 40
_SUP = 200
_NU = _SUP // _UNIT

_MM_BLK = 4000
_K = 5
_SEG = _N_EDGES // _K


def _make_gather_add(seg_base, seg_edges, interpret):
  epw = seg_edges // _NW
  nsup = epw // _SUP
  assert nsup * _SUP == epw and epw % 8 == 0

  mesh = plsc.VectorSubcoreMesh(
      core_axis_name="c", subcore_axis_name="s", num_cores=2, num_subcores=16)

  @functools.partial(
      pl.kernel,
      out_type=jax.ShapeDtypeStruct((seg_edges, _D), jnp.float32),
      mesh=mesh,
      scratch_types=[
          pltpu.VMEM((epw,), jnp.int32),
          pltpu.VMEM((epw,), jnp.int32),
          pltpu.VMEM((2, _SUP, _D), jnp.float32),
          pltpu.VMEM((2, _SUP, _D), jnp.float32),
          pltpu.SemaphoreType.DMA,
          pltpu.SemaphoreType.DMA,
          pltpu.SemaphoreType.DMA,
          pltpu.SemaphoreType.DMA,
          pltpu.SemaphoreType.DMA,
          pltpu.SemaphoreType.DMA,
      ],
      interpret=interpret,
  )
  def gather_add(pr_hbm, ps_hbm, recv_hbm, send_hbm, out_hbm,
                 idx_r, idx_s, rows_r, rows_s,
                 sem_r0, sem_s0, sem_r1, sem_s1, sem_o0, sem_o1):
    wid = lax.axis_index("s") * 2 + lax.axis_index("c")
    out_base = wid * epw
    in_base = seg_base + out_base
    sem_r = (sem_r0, sem_r1)
    sem_s = (sem_s0, sem_s1)
    sem_o = (sem_o0, sem_o1)

    pltpu.sync_copy(recv_hbm.at[pl.ds(in_base, epw)], idx_r)
    pltpu.sync_copy(send_hbm.at[pl.ds(in_base, epw)], idx_s)

    def fire(b, si):
      for u in range(_NU):
        o = si * _SUP + u * _UNIT
        dst = pl.ds(u * _UNIT, _UNIT)
        pltpu.async_copy(pr_hbm.at[idx_r.at[pl.ds(o, _UNIT)]],
                         rows_r.at[b, dst], sem_r[b])
        pltpu.async_copy(ps_hbm.at[idx_s.at[pl.ds(o, _UNIT)]],
                         rows_s.at[b, dst], sem_s[b])

    def drain(b, si):
      for u in range(_NU):
        o = si * _SUP + u * _UNIT
        dst = pl.ds(u * _UNIT, _UNIT)
        pltpu.make_async_copy(pr_hbm.at[idx_r.at[pl.ds(o, _UNIT)]],
                              rows_r.at[b, dst], sem_r[b]).wait()
        pltpu.make_async_copy(ps_hbm.at[idx_s.at[pl.ds(o, _UNIT)]],
                              rows_s.at[b, dst], sem_s[b]).wait()

    def consume(b, si):
      drain(b, si)

      def add_row(r, c2):
        for j in range(_D // 16):
          sl = pl.ds(j * 16, 16)
          plsc.addupdate(rows_r.at[b, r, sl], rows_s[b, r, sl])
        return c2

      lax.fori_loop(0, _SUP, add_row, 0)
      pltpu.async_copy(rows_r.at[b],
                       out_hbm.at[pl.ds(out_base + si * _SUP, _SUP)], sem_o[b])

    def wait_out(b, si):
      pltpu.make_async_copy(rows_r.at[b],
                            out_hbm.at[pl.ds(out_base + si * _SUP, _SUP)],
                            sem_o[b]).wait()

    fire(0, 0)
    fire(1, 1)

    def pair_body(i, carry):
      s0 = 2 * i
      consume(0, s0)

      @pl.when(s0 + 2 < nsup)
      def _():
        wait_out(0, s0)
        fire(0, s0 + 2)

      consume(1, s0 + 1)

      @pl.when(s0 + 3 < nsup)
      def _():
        wait_out(1, s0 + 1)
        fire(1, s0 + 3)

      return carry

    lax.fori_loop(0, nsup // 2, pair_body, 0)
    if nsup % 2:
      consume(0, nsup - 1)
      wait_out(0, nsup - 1)
      wait_out(1, nsup - 2)
    else:
      wait_out(0, nsup - 2)
      wait_out(1, nsup - 1)

  return gather_add


def _build(interpret: bool = False):
  def _proj_body(nodes_ref, wr_ref, ws_ref, pr_ref, ps_ref):
    n = nodes_ref[...]
    pr_ref[...] = jnp.dot(n, wr_ref[...], preferred_element_type=jnp.float32)
    ps_ref[...] = jnp.dot(n, ws_ref[...], preferred_element_type=jnp.float32)

  proj = pl.pallas_call(
      _proj_body,
      out_shape=(
          jax.ShapeDtypeStruct((_N_NODES, _D), jnp.float32),
          jax.ShapeDtypeStruct((_N_NODES, _D), jnp.float32),
      ),
      interpret=interpret,
  )

  scs = [_make_gather_add(k * _SEG, _SEG, interpret) for k in range(_K)]

  blocks = _SEG // _MM_BLK

  def _mm_body(e_ref, w_ref, g_ref, o_ref):
    o_ref[...] = (
        jnp.dot(e_ref[...], w_ref[...], preferred_element_type=jnp.float32)
        + g_ref[...]
    )

  def _mm_chain_body(e_ref, w_ref, g_ref, prev_ref, o_ref):
    del prev_ref
    o_ref[...] = (
        jnp.dot(e_ref[...], w_ref[...], preferred_element_type=jnp.float32)
        + g_ref[...]
    )

  mms = []
  for k in range(_K):
    e_spec = pl.BlockSpec((_MM_BLK, _D), lambda i, k0=k: (i + k0 * blocks, 0))
    w_spec = pl.BlockSpec((_D, _D), lambda i: (0, 0))
    g_spec = pl.BlockSpec((_MM_BLK, _D), lambda i: (i, 0))
    o_spec = pl.BlockSpec((_MM_BLK, _D), lambda i, k0=k: (i + k0 * blocks, 0))
    if k == 0:
      mms.append(pl.pallas_call(
          _mm_body,
          grid=(blocks,),
          in_specs=[e_spec, w_spec, g_spec],
          out_specs=o_spec,
          out_shape=jax.ShapeDtypeStruct((_N_EDGES, _D), jnp.float32),
          interpret=interpret,
      ))
    else:
      mms.append(pl.pallas_call(
          _mm_chain_body,
          grid=(blocks,),
          in_specs=[e_spec, w_spec, g_spec,
                    pl.BlockSpec(memory_space=pl.ANY)],
          out_specs=o_spec,
          out_shape=jax.ShapeDtypeStruct((_N_EDGES, _D), jnp.float32),
          input_output_aliases={3: 0},
          interpret=interpret,
      ))

  return proj, scs, mms


_CACHE = []


def kernel(nodes, edges, receivers, senders, W):
  if not _CACHE:
    _CACHE.append(_build(False))
  proj, scs, mms = _CACHE[0]
  we = W[:_D]
  wr = W[_D:2 * _D]
  ws = W[2 * _D:]
  pr, ps = proj(nodes, wr, ws)
  recv = receivers.astype(jnp.int32)
  send = senders.astype(jnp.int32)
  gs = [sc(pr, ps, recv, send) for sc in scs]
  out = mms[0](edges, we, gs[0])
  for k in range(1, _K):
    out = mms[k](edges, we, gs[k], out)
  return out
